# baseline (speedup 1.0000x reference)
import jax
import jax.numpy as jnp
import numpy as np
from jax import lax
from jax.experimental import pallas as pl
from jax.experimental.pallas import tpu as pltpu
from jax.experimental.pallas import tpu_sc as plsc

_THRESHOLD = np.float32(5e-8)
_SCALE = np.float32(1e-7)
_FIX = np.float32(2.0**16)
_UNFIX = np.float32(2.0**-16)

_BATCH = 16384
_LANES = 16
_NUM_TILES = 16
_PER_TILE = _BATCH // _NUM_TILES
_CHUNKS = _PER_TILE // _LANES

_mesh = plsc.VectorSubcoreMesh(
    core_axis_name="c", subcore_axis_name="s", num_cores=1
)


def _masked_sum_body(x_hbm, out_hbm, slice_v, part_v, acc_smem):
    sid = lax.axis_index("s")

    @pl.when(sid == 0)
    def _init():
        acc_smem[0] = jnp.int32(0)

    plsc.subcore_barrier()

    base = sid * _PER_TILE
    pltpu.sync_copy(x_hbm.at[pl.ds(base, _PER_TILE)], slice_v)

    acc = jnp.zeros((_LANES,), jnp.float32)
    for i in range(_CHUNKS):
        x = slice_v[pl.ds(i * _LANES, _LANES)]
        keep = (x * _SCALE) < _THRESHOLD
        acc = acc + jnp.where(keep, x, np.float32(0.0))
    ids = lax.iota(jnp.int32, _LANES)
    for shift in (1, 2, 4, 8):
        acc = acc + acc.at[ids ^ shift].get(mode="promise_in_bounds")
    q = (acc * _FIX + np.float32(0.5)).astype(jnp.int32)
    plsc.fetch_and_add(acc_smem.at[0], q[0], subcore_id=0)
    plsc.subcore_barrier()

    @pl.when(sid == 0)
    def _finalize():
        total_q = jnp.full((_LANES,), acc_smem[0], jnp.int32)
        part_v[...] = total_q.astype(jnp.float32) * _UNFIX
        pltpu.sync_copy(part_v, out_hbm)


_masked_sum_sc = pl.kernel(
    _masked_sum_body,
    out_type=jax.ShapeDtypeStruct((_LANES,), jnp.float32),
    mesh=_mesh,
    scratch_types=[
        pltpu.VMEM((_PER_TILE,), jnp.float32),
        pltpu.VMEM((_LANES,), jnp.float32),
        pltpu.SMEM((8,), jnp.int32),
    ],
)


def kernel(super_loss, index, v):
    del index, v
    out = _masked_sum_sc(super_loss)
    return out[0]


# speedup vs baseline: 1.0010x; 1.0010x over previous
import jax
import jax.numpy as jnp
import numpy as np
from jax import lax
from jax.experimental import pallas as pl
from jax.experimental.pallas import tpu as pltpu
from jax.experimental.pallas import tpu_sc as plsc

_THRESHOLD = np.float32(5e-8)
_SCALE = np.float32(1e-7)
_FIX = np.float32(2.0**16)
_UNFIX = np.float32(2.0**-16)

_BATCH = 16384
_LANES = 16
_NUM_TILES = 16
_PER_TILE = _BATCH // _NUM_TILES
_CHUNKS = _PER_TILE // _LANES

_mesh = plsc.VectorSubcoreMesh(
    core_axis_name="c", subcore_axis_name="s", num_cores=1
)


def _masked_sum_body(x_hbm, out_hbm, slice_v, part_v, acc_smem, sem0, sem1):
    sid = lax.axis_index("s")
    base = sid * _PER_TILE
    half = _PER_TILE // 2

    cp0 = pltpu.async_copy(
        x_hbm.at[pl.ds(base, half)], slice_v.at[pl.ds(0, half)], sem0
    )
    cp1 = pltpu.async_copy(
        x_hbm.at[pl.ds(base + half, half)], slice_v.at[pl.ds(half, half)], sem1
    )

    @pl.when(sid == 0)
    def _init():
        acc_smem[0] = jnp.int32(0)

    plsc.subcore_barrier()

    acc = jnp.zeros((_LANES,), jnp.float32)
    cp0.wait()
    for i in range(_CHUNKS // 2):
        x = slice_v[pl.ds(i * _LANES, _LANES)]
        keep = (x * _SCALE) < _THRESHOLD
        acc = acc + jnp.where(keep, x, np.float32(0.0))
    cp1.wait()
    for i in range(_CHUNKS // 2, _CHUNKS):
        x = slice_v[pl.ds(i * _LANES, _LANES)]
        keep = (x * _SCALE) < _THRESHOLD
        acc = acc + jnp.where(keep, x, np.float32(0.0))
    ids = lax.iota(jnp.int32, _LANES)
    for shift in (1, 2, 4, 8):
        acc = acc + acc.at[ids ^ shift].get(mode="promise_in_bounds")
    q = (acc * _FIX + np.float32(0.5)).astype(jnp.int32)
    plsc.fetch_and_add(acc_smem.at[0], q[0], subcore_id=0)
    plsc.subcore_barrier()

    @pl.when(sid == 0)
    def _finalize():
        total_q = jnp.full((_LANES,), acc_smem[0], jnp.int32)
        part_v[...] = total_q.astype(jnp.float32) * _UNFIX
        pltpu.sync_copy(part_v, out_hbm)


_masked_sum_sc = pl.kernel(
    _masked_sum_body,
    out_type=jax.ShapeDtypeStruct((_LANES,), jnp.float32),
    mesh=_mesh,
    scratch_types=[
        pltpu.VMEM((_PER_TILE,), jnp.float32),
        pltpu.VMEM((_LANES,), jnp.float32),
        pltpu.SMEM((8,), jnp.int32),
        pltpu.SemaphoreType.DMA,
        pltpu.SemaphoreType.DMA,
    ],
)


def kernel(super_loss, index, v):
    del index, v
    out = _masked_sum_sc(super_loss)
    return out[0]


# speedup vs baseline: 1.0131x; 1.0120x over previous
import jax
import jax.numpy as jnp
import numpy as np
from jax import lax
from jax.experimental import pallas as pl
from jax.experimental.pallas import tpu as pltpu
from jax.experimental.pallas import tpu_sc as plsc

_THRESHOLD = np.float32(5e-8)
_SCALE = np.float32(1e-7)
_FIX = np.float32(2.0**16)
_UNFIX = np.float32(2.0**-16)

_BATCH = 16384
_LANES = 16
_NUM_TILES = 16
_PER_TILE = _BATCH // _NUM_TILES
_CHUNKS = _PER_TILE // _LANES

_mesh = plsc.VectorSubcoreMesh(
    core_axis_name="c", subcore_axis_name="s", num_cores=1
)


def _masked_sum_body(x_hbm, out_hbm, slice_v, part_v, acc_smem):
    sid = lax.axis_index("s")

    @pl.when(sid == 0)
    def _init():
        acc_smem[0] = jnp.int32(0)

    plsc.subcore_barrier()

    base = sid * _PER_TILE
    pltpu.sync_copy(x_hbm.at[pl.ds(base, _PER_TILE)], slice_v)

    acc = jnp.zeros((_LANES,), jnp.float32)
    for i in range(_CHUNKS):
        x = slice_v[pl.ds(i * _LANES, _LANES)]
        keep = (x * _SCALE) < _THRESHOLD
        acc = acc + jnp.where(keep, x, np.float32(0.0))
    ids = lax.iota(jnp.int32, _LANES)
    for shift in (1, 2, 4, 8):
        acc = acc + acc.at[ids ^ shift].get(mode="promise_in_bounds")
    q = (acc * _FIX + np.float32(0.5)).astype(jnp.int32)
    plsc.fetch_and_add(acc_smem.at[0], q[0], subcore_id=0)
    plsc.subcore_barrier()

    @pl.when(sid == 0)
    def _finalize():
        total_q = jnp.full((_LANES,), acc_smem[0], jnp.int32)
        part_v[...] = total_q.astype(jnp.float32) * _UNFIX
        pltpu.sync_copy(part_v, out_hbm)


_masked_sum_sc = pl.kernel(
    _masked_sum_body,
    out_type=jax.ShapeDtypeStruct((_LANES,), jnp.float32),
    mesh=_mesh,
    scratch_types=[
        pltpu.VMEM((_PER_TILE,), jnp.float32),
        pltpu.VMEM((_LANES,), jnp.float32),
        pltpu.SMEM((8,), jnp.int32),
    ],
)


def kernel(super_loss, index, v):
    del index, v
    out = _masked_sum_sc(super_loss)
    return out[0]
